# baseline (speedup 1.0000x reference)
import functools

import jax
import jax.numpy as jnp
from jax import lax
from jax.experimental import pallas as pl
from jax.experimental.pallas import tpu as pltpu
from jax.experimental.pallas import tpu_sc as plsc

_N = 10000
_D = 128
_DO = 64
_E = 320000
_NPAD = 10240
_CHUNK = 128
_EROWS = 2560
_EPAD = _EROWS * _CHUNK
_TPT = _EROWS // 16
_TPT2 = _EROWS // 32
_RPT = _NPAD // 16

@functools.cache
def _mesh():
    return plsc.VectorSubcoreMesh(
        core_axis_name="c", subcore_axis_name="s",
        num_cores=2, num_subcores=16)


def _deg_body(dst_hbm, out_hbm, dst_v, deg_v, tmp_v, seg_v, parts):
    c = lax.axis_index("c")
    s = lax.axis_index("s")

    @pl.loop(0, _NPAD // 16)
    def _zero(i):
        deg_v[pl.ds(i * 16, 16)] = jnp.zeros((16,), jnp.float32)

    pltpu.sync_copy(dst_hbm.at[pl.ds(c * (_EROWS // 2) + s * _TPT2, _TPT2)],
                    dst_v)
    ones = jnp.full((16,), 1.0, jnp.float32)

    @pl.loop(0, _TPT2)
    def _count(j):
        for k in range(_CHUNK // 16):
            vec = dst_v[j, pl.ds(k * 16, 16)]
            plsc.addupdate_scatter(deg_v, [vec], ones)

    pltpu.sync_copy(deg_v, parts.at[s])
    plsc.subcore_barrier()

    @pl.loop(0, _RPT // 16)
    def _zero2(i):
        seg_v[pl.ds(i * 16, 16)] = jnp.zeros((16,), jnp.float32)

    @pl.loop(0, 16)
    def _reduce(p):
        pltpu.sync_copy(parts.at[p, pl.ds(s * _RPT, _RPT)], tmp_v)

        @pl.loop(0, _RPT // 16)
        def _add(i):
            sl = pl.ds(i * 16, 16)
            seg_v[sl] = seg_v[sl] + tmp_v[sl]

    pltpu.sync_copy(seg_v, out_hbm.at[c, pl.ds(s * _RPT, _RPT)])


@functools.cache
def _deg_kernel():
    return pl.kernel(
        _deg_body,
        out_type=jax.ShapeDtypeStruct((2, _NPAD), jnp.float32),
        mesh=_mesh(),
        compiler_params=pltpu.CompilerParams(needs_layout_passes=False),
        scratch_types=[
            pltpu.VMEM((_TPT2, _CHUNK), jnp.int32),
            pltpu.VMEM((_NPAD,), jnp.float32),
            pltpu.VMEM((_RPT,), jnp.float32),
            pltpu.VMEM((_RPT,), jnp.float32),
            pltpu.VMEM_SHARED((16, _NPAD), jnp.float32),
        ],
    )


def _deg(dstp):
    return _deg_kernel()(dstp)


_GRP = 16
_NGRP = _TPT2 // _GRP


def _spmm_body(tab_hbm, src_hbm, dst_hbm, out_hbm,
               sidx, didx, rows0, rows1, acc,
               sem_i, sem_g0, sem_g1, sem_s0, sem_s1):
    c = lax.axis_index("c")
    s = lax.axis_index("s")

    @pl.loop(0, _CHUNK)
    def _fill(i):
        for k in range(_D // 16):
            rows0[i, pl.ds(k * 16, 16)] = jnp.zeros((16,), jnp.float32)

    for k in range(_RPT // _CHUNK):
        pltpu.sync_copy(rows0, acc.at[pl.ds(s * _RPT + k * _CHUNK, _CHUNK)])

    base = (1 - c) * (_EROWS // 2) + s * _TPT2
    pltpu.sync_copy(src_hbm.at[pl.ds(base, _GRP)], sidx.at[0])
    pltpu.sync_copy(dst_hbm.at[pl.ds(base, _GRP)], didx.at[0])

    plsc.subcore_barrier()

    pltpu.async_copy(tab_hbm.at[sidx.at[0, 0]], rows0, sem_g0)
    for g in range(_NGRP):
        bg = g % 2
        if g + 1 < _NGRP:
            nb = (g + 1) % 2
            pltpu.async_copy(
                src_hbm.at[pl.ds(base + (g + 1) * _GRP, _GRP)],
                sidx.at[nb], sem_i)
            pltpu.async_copy(
                dst_hbm.at[pl.ds(base + (g + 1) * _GRP, _GRP)],
                didx.at[nb], sem_i)

        @pl.loop(0, _GRP, step=2)
        def _pipe(j):
            def _wait_s1():
                pltpu.make_async_copy(
                    rows1, acc.at[didx.at[bg, 0]], sem_s1).wait()
            if g == 0:
                @pl.when(j > 0)
                def _():
                    _wait_s1()
            else:
                _wait_s1()
            pltpu.async_copy(tab_hbm.at[sidx.at[bg, j + 1]], rows1, sem_g1)

            pltpu.make_async_copy(
                tab_hbm.at[sidx.at[bg, j]], rows0, sem_g0).wait()
            pltpu.async_copy(rows0, acc.at[didx.at[bg, j]], sem_s0, add=True)

            pltpu.make_async_copy(
                tab_hbm.at[sidx.at[bg, j + 1]], rows1, sem_g1).wait()
            pltpu.async_copy(rows1, acc.at[didx.at[bg, j + 1]], sem_s1,
                             add=True)

            pltpu.make_async_copy(
                rows0, acc.at[didx.at[bg, 0]], sem_s0).wait()

            @pl.when(j + 2 < _GRP)
            def _():
                pltpu.async_copy(
                    tab_hbm.at[sidx.at[bg, j + 2]], rows0, sem_g0)

        if g + 1 < _NGRP:
            nb = (g + 1) % 2
            pltpu.make_async_copy(
                src_hbm.at[pl.ds(base + (g + 1) * _GRP, _GRP)],
                sidx.at[nb], sem_i).wait()
            pltpu.make_async_copy(
                dst_hbm.at[pl.ds(base + (g + 1) * _GRP, _GRP)],
                didx.at[nb], sem_i).wait()
            pltpu.async_copy(tab_hbm.at[sidx.at[nb, 0]], rows0, sem_g0)

    pltpu.make_async_copy(
        rows1, acc.at[didx.at[(_NGRP - 1) % 2, 0]], sem_s1).wait()
    plsc.subcore_barrier()
    pltpu.sync_copy(acc.at[pl.ds(s * _RPT, _RPT)],
                    out_hbm.at[c, pl.ds(s * _RPT, _RPT)])


@functools.cache
def _spmm_kernel():
    return pl.kernel(
        _spmm_body,
        out_type=jax.ShapeDtypeStruct((2, _NPAD, _D), jnp.float32),
        mesh=_mesh(),
        scratch_types=[
            pltpu.VMEM((2, _GRP, _CHUNK), jnp.int32),
            pltpu.VMEM((2, _GRP, _CHUNK), jnp.int32),
            pltpu.VMEM((_CHUNK, _D), jnp.float32),
            pltpu.VMEM((_CHUNK, _D), jnp.float32),
            pltpu.VMEM_SHARED((_NPAD, _D), jnp.float32),
            pltpu.SemaphoreType.DMA,
            pltpu.SemaphoreType.DMA,
            pltpu.SemaphoreType.DMA,
            pltpu.SemaphoreType.DMA,
            pltpu.SemaphoreType.DMA,
        ],
    )


def _spmm(tab, srcp, dstp):
    return _spmm_kernel()(tab, srcp, dstp)


_BN = 2000


def _tca_body(x_ref, w1_ref, d0_ref, d1_ref, y_ref, dis_ref):
    xw = jnp.dot(x_ref[...], w1_ref[...], preferred_element_type=jnp.float32)
    dis = lax.rsqrt(d0_ref[...] + d1_ref[...] + 1.0)
    y_ref[...] = xw * dis
    dis_ref[...] = dis


def _tca(x, w1, d0, d1):
    return pl.pallas_call(
        _tca_body,
        grid=(_N // _BN,),
        in_specs=[
            pl.BlockSpec((_BN, _D), lambda i: (i, 0)),
            pl.BlockSpec((_D, _D), lambda i: (0, 0)),
            pl.BlockSpec((_BN, 1), lambda i: (i, 0)),
            pl.BlockSpec((_BN, 1), lambda i: (i, 0)),
        ],
        out_specs=[
            pl.BlockSpec((_BN, _D), lambda i: (i, 0)),
            pl.BlockSpec((_BN, 1), lambda i: (i, 0)),
        ],
        out_shape=[
            jax.ShapeDtypeStruct((_N, _D), jnp.float32),
            jax.ShapeDtypeStruct((_N, 1), jnp.float32),
        ],
    )(x, w1, d0, d1)


def _tcb_body(s_ref, y_ref, dis_ref, b1_ref, y2_ref):
    dis = dis_ref[...]
    h = jnp.maximum(dis * (s_ref[0] + s_ref[1] + y_ref[...]) + b1_ref[...],
                    0.0)
    y2_ref[...] = dis * h


def _tcb(s1, y, dis, b1):
    return pl.pallas_call(
        _tcb_body,
        grid=(_N // _BN,),
        in_specs=[
            pl.BlockSpec((2, _BN, _D), lambda i: (0, i, 0)),
            pl.BlockSpec((_BN, _D), lambda i: (i, 0)),
            pl.BlockSpec((_BN, 1), lambda i: (i, 0)),
            pl.BlockSpec((1, _D), lambda i: (0, 0)),
        ],
        out_specs=pl.BlockSpec((_BN, _D), lambda i: (i, 0)),
        out_shape=jax.ShapeDtypeStruct((_N, _D), jnp.float32),
    )(s1, y, dis, b1)


def _tcc_body(s_ref, y2_ref, dis_ref, wmu_ref, bmu_ref, wls_ref, bls_ref,
              mu_ref, ls_ref):
    agg = dis_ref[...] * (s_ref[0] + s_ref[1] + y2_ref[...])
    mu_ref[...] = (jnp.dot(agg, wmu_ref[...],
                           preferred_element_type=jnp.float32) + bmu_ref[...])
    ls_ref[...] = (jnp.dot(agg, wls_ref[...],
                           preferred_element_type=jnp.float32) + bls_ref[...])


def _tcc(s2, y2, dis, wmu, bmu, wls, bls):
    return pl.pallas_call(
        _tcc_body,
        grid=(_N // _BN,),
        in_specs=[
            pl.BlockSpec((2, _BN, _D), lambda i: (0, i, 0)),
            pl.BlockSpec((_BN, _D), lambda i: (i, 0)),
            pl.BlockSpec((_BN, 1), lambda i: (i, 0)),
            pl.BlockSpec((_D, _DO), lambda i: (0, 0)),
            pl.BlockSpec((1, _DO), lambda i: (0, 0)),
            pl.BlockSpec((_D, _DO), lambda i: (0, 0)),
            pl.BlockSpec((1, _DO), lambda i: (0, 0)),
        ],
        out_specs=[
            pl.BlockSpec((_BN, _DO), lambda i: (i, 0)),
            pl.BlockSpec((_BN, _DO), lambda i: (i, 0)),
        ],
        out_shape=[
            jax.ShapeDtypeStruct((_N, _DO), jnp.float32),
            jax.ShapeDtypeStruct((_N, _DO), jnp.float32),
        ],
    )(s2, y2, dis, wmu, bmu, wls, bls)


def kernel(x, edge_index, W1, b1, Wmu, bmu, Wls, bls):
    src = edge_index[0]
    dst = edge_index[1]
    pad = _EPAD - _E
    srcp = jnp.concatenate(
        [src, jnp.zeros((pad,), jnp.int32)]).reshape(_EROWS, _CHUNK)
    dstp = jnp.concatenate(
        [dst, _N + (jnp.arange(pad, dtype=jnp.int32) % (_NPAD - _N))]
    ).reshape(_EROWS, _CHUNK)

    dcnt = _deg(dstp)
    d0 = dcnt[0, :_N].reshape(_N, 1)
    d1 = dcnt[1, :_N].reshape(_N, 1)

    y, dis = _tca(x, W1, d0, d1)
    s1 = _spmm(y, srcp, dstp)[:, :_N, :]
    y2 = _tcb(s1, y, dis, b1.reshape(1, _D))
    s2 = _spmm(y2, srcp, dstp)[:, :_N, :]
    mu, ls = _tcc(s2, y2, dis, Wmu, bmu.reshape(1, _DO), Wls,
                  bls.reshape(1, _DO))
    return (mu, ls)


# speedup vs baseline: 1.0698x; 1.0698x over previous
import functools

import jax
import jax.numpy as jnp
from jax import lax
from jax.experimental import pallas as pl
from jax.experimental.pallas import tpu as pltpu
from jax.experimental.pallas import tpu_sc as plsc

_N = 10000
_D = 128
_DO = 64
_E = 320000
_NPAD = 10240
_CHUNK = 128
_EROWS = 2560
_EPAD = _EROWS * _CHUNK
_TPT = _EROWS // 16
_TPT2 = _EROWS // 32
_RPT = _NPAD // 16

@functools.cache
def _mesh():
    return plsc.VectorSubcoreMesh(
        core_axis_name="c", subcore_axis_name="s",
        num_cores=2, num_subcores=16)


def _deg_body(dst_hbm, out_hbm, dst_v, deg_v, tmp_v, seg_v, parts):
    c = lax.axis_index("c")
    s = lax.axis_index("s")

    @pl.loop(0, _NPAD // 16)
    def _zero(i):
        deg_v[pl.ds(i * 16, 16)] = jnp.zeros((16,), jnp.float32)

    pltpu.sync_copy(dst_hbm.at[pl.ds(c * (_EROWS // 2) + s * _TPT2, _TPT2)],
                    dst_v)
    ones = jnp.full((16,), 1.0, jnp.float32)

    @pl.loop(0, _TPT2)
    def _count(j):
        for k in range(_CHUNK // 16):
            vec = dst_v[j, pl.ds(k * 16, 16)]
            plsc.addupdate_scatter(deg_v, [vec], ones)

    pltpu.sync_copy(deg_v, parts.at[s])
    plsc.subcore_barrier()

    @pl.loop(0, _RPT // 16)
    def _zero2(i):
        seg_v[pl.ds(i * 16, 16)] = jnp.zeros((16,), jnp.float32)

    @pl.loop(0, 16)
    def _reduce(p):
        pltpu.sync_copy(parts.at[p, pl.ds(s * _RPT, _RPT)], tmp_v)

        @pl.loop(0, _RPT // 16)
        def _add(i):
            sl = pl.ds(i * 16, 16)
            seg_v[sl] = seg_v[sl] + tmp_v[sl]

    pltpu.sync_copy(seg_v, out_hbm.at[c, pl.ds(s * _RPT, _RPT)])


@functools.cache
def _deg_kernel():
    return pl.kernel(
        _deg_body,
        out_type=jax.ShapeDtypeStruct((2, _NPAD), jnp.float32),
        mesh=_mesh(),
        compiler_params=pltpu.CompilerParams(needs_layout_passes=False),
        scratch_types=[
            pltpu.VMEM((_TPT2, _CHUNK), jnp.int32),
            pltpu.VMEM((_NPAD,), jnp.float32),
            pltpu.VMEM((_RPT,), jnp.float32),
            pltpu.VMEM((_RPT,), jnp.float32),
            pltpu.VMEM_SHARED((16, _NPAD), jnp.float32),
        ],
    )


def _deg(dstp):
    return _deg_kernel()(dstp)


_C0_TPT = 120
_C1_TPT = 40
_C0_GRP = 24
_C1_GRP = 8
_NGRP = 5
_GRPMAX = 24


def _edge_pipeline(tab_hbm, src_hbm, dst_hbm, sidx, didx, rows0, rows1, acc,
                   sem_i, sem_g0, sem_g1, base, grp):
    pltpu.sync_copy(src_hbm.at[pl.ds(base, grp)], sidx.at[0, pl.ds(0, grp)])
    pltpu.sync_copy(dst_hbm.at[pl.ds(base, grp)], didx.at[0, pl.ds(0, grp)])
    pltpu.async_copy(tab_hbm.at[sidx.at[0, 0]], rows0, sem_g0)
    for g in range(_NGRP):
        bg = g % 2
        if g + 1 < _NGRP:
            nb = (g + 1) % 2
            pltpu.async_copy(
                src_hbm.at[pl.ds(base + (g + 1) * grp, grp)],
                sidx.at[nb, pl.ds(0, grp)], sem_i)
            pltpu.async_copy(
                dst_hbm.at[pl.ds(base + (g + 1) * grp, grp)],
                didx.at[nb, pl.ds(0, grp)], sem_i)

        @pl.loop(0, grp, step=2)
        def _pipe(j):
            pltpu.async_copy(tab_hbm.at[sidx.at[bg, j + 1]], rows1, sem_g1)
            pltpu.make_async_copy(
                tab_hbm.at[sidx.at[bg, j]], rows0, sem_g0).wait()
            pltpu.sync_copy(rows0, acc.at[didx.at[bg, j]], add=True)

            @pl.when(j + 2 < grp)
            def _():
                pltpu.async_copy(
                    tab_hbm.at[sidx.at[bg, j + 2]], rows0, sem_g0)

            pltpu.make_async_copy(
                tab_hbm.at[sidx.at[bg, j + 1]], rows1, sem_g1).wait()
            pltpu.sync_copy(rows1, acc.at[didx.at[bg, j + 1]], add=True)

        if g + 1 < _NGRP:
            nb = (g + 1) % 2
            pltpu.make_async_copy(
                src_hbm.at[pl.ds(base + (g + 1) * grp, grp)],
                sidx.at[nb, pl.ds(0, grp)], sem_i).wait()
            pltpu.make_async_copy(
                dst_hbm.at[pl.ds(base + (g + 1) * grp, grp)],
                didx.at[nb, pl.ds(0, grp)], sem_i).wait()
            pltpu.async_copy(tab_hbm.at[sidx.at[nb, 0]], rows0, sem_g0)


def _spmm_body(tab_hbm, src_hbm, dst_hbm, out_hbm,
               sidx, didx, rows0, rows1, acc,
               sem_i, sem_g0, sem_g1):
    c = lax.axis_index("c")
    s = lax.axis_index("s")

    @pl.loop(0, _CHUNK)
    def _fill(i):
        for k in range(_D // 16):
            rows0[i, pl.ds(k * 16, 16)] = jnp.zeros((16,), jnp.float32)

    for k in range(_RPT // _CHUNK):
        pltpu.sync_copy(rows0, acc.at[pl.ds(s * _RPT + k * _CHUNK, _CHUNK)])

    plsc.subcore_barrier()

    @pl.when(c == 0)
    def _core0():
        _edge_pipeline(tab_hbm, src_hbm, dst_hbm, sidx, didx, rows0, rows1,
                       acc, sem_i, sem_g0, sem_g1,
                       base=s * _C0_TPT, grp=_C0_GRP)

    @pl.when(c == 1)
    def _core1():
        _edge_pipeline(tab_hbm, src_hbm, dst_hbm, sidx, didx, rows0, rows1,
                       acc, sem_i, sem_g0, sem_g1,
                       base=16 * _C0_TPT + s * _C1_TPT, grp=_C1_GRP)

    plsc.subcore_barrier()
    pltpu.sync_copy(acc.at[pl.ds(s * _RPT, _RPT)],
                    out_hbm.at[c, pl.ds(s * _RPT, _RPT)])


@functools.cache
def _spmm_kernel():
    return pl.kernel(
        _spmm_body,
        out_type=jax.ShapeDtypeStruct((2, _NPAD, _D), jnp.float32),
        mesh=_mesh(),
        scratch_types=[
            pltpu.VMEM((2, _GRPMAX, _CHUNK), jnp.int32),
            pltpu.VMEM((2, _GRPMAX, _CHUNK), jnp.int32),
            pltpu.VMEM((_CHUNK, _D), jnp.float32),
            pltpu.VMEM((_CHUNK, _D), jnp.float32),
            pltpu.VMEM_SHARED((_NPAD, _D), jnp.float32),
            pltpu.SemaphoreType.DMA,
            pltpu.SemaphoreType.DMA,
            pltpu.SemaphoreType.DMA,
        ],
    )


def _spmm(tab, srcp, dstp):
    return _spmm_kernel()(tab, srcp, dstp)


_BN = 2000


def _tca_body(x_ref, w1_ref, d0_ref, d1_ref, y_ref, dis_ref):
    xw = jnp.dot(x_ref[...], w1_ref[...], preferred_element_type=jnp.float32)
    dis = lax.rsqrt(d0_ref[...] + d1_ref[...] + 1.0)
    y_ref[...] = xw * dis
    dis_ref[...] = dis


def _tca(x, w1, d0, d1):
    return pl.pallas_call(
        _tca_body,
        grid=(_N // _BN,),
        in_specs=[
            pl.BlockSpec((_BN, _D), lambda i: (i, 0)),
            pl.BlockSpec((_D, _D), lambda i: (0, 0)),
            pl.BlockSpec((_BN, 1), lambda i: (i, 0)),
            pl.BlockSpec((_BN, 1), lambda i: (i, 0)),
        ],
        out_specs=[
            pl.BlockSpec((_BN, _D), lambda i: (i, 0)),
            pl.BlockSpec((_BN, 1), lambda i: (i, 0)),
        ],
        out_shape=[
            jax.ShapeDtypeStruct((_N, _D), jnp.float32),
            jax.ShapeDtypeStruct((_N, 1), jnp.float32),
        ],
    )(x, w1, d0, d1)


def _tcb_body(s_ref, y_ref, dis_ref, b1_ref, y2_ref):
    dis = dis_ref[...]
    h = jnp.maximum(dis * (s_ref[0] + s_ref[1] + y_ref[...]) + b1_ref[...],
                    0.0)
    y2_ref[...] = dis * h


def _tcb(s1, y, dis, b1):
    return pl.pallas_call(
        _tcb_body,
        grid=(_N // _BN,),
        in_specs=[
            pl.BlockSpec((2, _BN, _D), lambda i: (0, i, 0)),
            pl.BlockSpec((_BN, _D), lambda i: (i, 0)),
            pl.BlockSpec((_BN, 1), lambda i: (i, 0)),
            pl.BlockSpec((1, _D), lambda i: (0, 0)),
        ],
        out_specs=pl.BlockSpec((_BN, _D), lambda i: (i, 0)),
        out_shape=jax.ShapeDtypeStruct((_N, _D), jnp.float32),
    )(s1, y, dis, b1)


def _tcc_body(s_ref, y2_ref, dis_ref, wmu_ref, bmu_ref, wls_ref, bls_ref,
              mu_ref, ls_ref):
    agg = dis_ref[...] * (s_ref[0] + s_ref[1] + y2_ref[...])
    mu_ref[...] = (jnp.dot(agg, wmu_ref[...],
                           preferred_element_type=jnp.float32) + bmu_ref[...])
    ls_ref[...] = (jnp.dot(agg, wls_ref[...],
                           preferred_element_type=jnp.float32) + bls_ref[...])


def _tcc(s2, y2, dis, wmu, bmu, wls, bls):
    return pl.pallas_call(
        _tcc_body,
        grid=(_N // _BN,),
        in_specs=[
            pl.BlockSpec((2, _BN, _D), lambda i: (0, i, 0)),
            pl.BlockSpec((_BN, _D), lambda i: (i, 0)),
            pl.BlockSpec((_BN, 1), lambda i: (i, 0)),
            pl.BlockSpec((_D, _DO), lambda i: (0, 0)),
            pl.BlockSpec((1, _DO), lambda i: (0, 0)),
            pl.BlockSpec((_D, _DO), lambda i: (0, 0)),
            pl.BlockSpec((1, _DO), lambda i: (0, 0)),
        ],
        out_specs=[
            pl.BlockSpec((_BN, _DO), lambda i: (i, 0)),
            pl.BlockSpec((_BN, _DO), lambda i: (i, 0)),
        ],
        out_shape=[
            jax.ShapeDtypeStruct((_N, _DO), jnp.float32),
            jax.ShapeDtypeStruct((_N, _DO), jnp.float32),
        ],
    )(s2, y2, dis, wmu, bmu, wls, bls)


def kernel(x, edge_index, W1, b1, Wmu, bmu, Wls, bls):
    src = edge_index[0]
    dst = edge_index[1]
    pad = _EPAD - _E
    srcp = jnp.concatenate(
        [src, jnp.zeros((pad,), jnp.int32)]).reshape(_EROWS, _CHUNK)
    dstp = jnp.concatenate(
        [dst, _N + (jnp.arange(pad, dtype=jnp.int32) % (_NPAD - _N))]
    ).reshape(_EROWS, _CHUNK)

    dcnt = _deg(dstp)
    d0 = dcnt[0, :_N].reshape(_N, 1)
    d1 = dcnt[1, :_N].reshape(_N, 1)

    y, dis = _tca(x, W1, d0, d1)
    s1 = _spmm(y, srcp, dstp)[:, :_N, :]
    y2 = _tcb(s1, y, dis, b1.reshape(1, _D))
    s2 = _spmm(y2, srcp, dstp)[:, :_N, :]
    mu, ls = _tcc(s2, y2, dis, Wmu, bmu.reshape(1, _DO), Wls,
                  bls.reshape(1, _DO))
    return (mu, ls)


# speedup vs baseline: 1.0699x; 1.0002x over previous
import functools

import jax
import jax.numpy as jnp
from jax import lax
from jax.experimental import pallas as pl
from jax.experimental.pallas import tpu as pltpu
from jax.experimental.pallas import tpu_sc as plsc

_N = 10000
_D = 128
_DO = 64
_E = 320000
_NPAD = 10240
_CHUNK = 128
_EROWS = 2560
_EPAD = _EROWS * _CHUNK
_TPT2 = _EROWS // 32
_RPT = _NPAD // 16

@functools.cache
def _mesh():
    return plsc.VectorSubcoreMesh(
        core_axis_name="c", subcore_axis_name="s",
        num_cores=2, num_subcores=16)


def _deg_body(dst_hbm, out_hbm, dst_v, deg_v, tmp_v, seg_v, parts):
    c = lax.axis_index("c")
    s = lax.axis_index("s")

    @pl.loop(0, _NPAD // 16)
    def _zero(i):
        deg_v[pl.ds(i * 16, 16)] = jnp.zeros((16,), jnp.float32)

    pltpu.sync_copy(dst_hbm.at[pl.ds(c * (_EROWS // 2) + s * _TPT2, _TPT2)],
                    dst_v)
    ones = jnp.full((16,), 1.0, jnp.float32)

    @pl.loop(0, _TPT2)
    def _count(j):
        for k in range(_CHUNK // 16):
            vec = dst_v[j, pl.ds(k * 16, 16)]
            plsc.addupdate_scatter(deg_v, [vec], ones)

    pltpu.sync_copy(deg_v, parts.at[s])
    plsc.subcore_barrier()

    @pl.loop(0, _RPT // 16)
    def _zero2(i):
        seg_v[pl.ds(i * 16, 16)] = jnp.zeros((16,), jnp.float32)

    @pl.loop(0, 16)
    def _reduce(p):
        pltpu.sync_copy(parts.at[p, pl.ds(s * _RPT, _RPT)], tmp_v)

        @pl.loop(0, _RPT // 16)
        def _add(i):
            sl = pl.ds(i * 16, 16)
            seg_v[sl] = seg_v[sl] + tmp_v[sl]

    pltpu.sync_copy(seg_v, out_hbm.at[c, pl.ds(s * _RPT, _RPT)])


@functools.cache
def _deg_kernel():
    return pl.kernel(
        _deg_body,
        out_type=jax.ShapeDtypeStruct((2, _NPAD), jnp.float32),
        mesh=_mesh(),
        compiler_params=pltpu.CompilerParams(needs_layout_passes=False),
        scratch_types=[
            pltpu.VMEM((_TPT2, _CHUNK), jnp.int32),
            pltpu.VMEM((_NPAD,), jnp.float32),
            pltpu.VMEM((_RPT,), jnp.float32),
            pltpu.VMEM((_RPT,), jnp.float32),
            pltpu.VMEM_SHARED((16, _NPAD), jnp.float32),
        ],
    )


def _deg(dstp):
    return _deg_kernel()(dstp)


_C0_TPT = 120
_C1_TPT = 40
_C0_GRP = 24
_C1_GRP = 8
_NGRP = 5
_GRPMAX = 24


def _edge_pipeline(tab_hbm, src_hbm, dst_hbm, sidx, didx, rows0, rows1, acc,
                   sem_i, sem_g0, sem_g1, base, grp):
    pltpu.sync_copy(src_hbm.at[pl.ds(base, grp)], sidx.at[0, pl.ds(0, grp)])
    pltpu.sync_copy(dst_hbm.at[pl.ds(base, grp)], didx.at[0, pl.ds(0, grp)])
    pltpu.async_copy(tab_hbm.at[sidx.at[0, 0]], rows0, sem_g0)
    for g in range(_NGRP):
        bg = g % 2
        if g + 1 < _NGRP:
            nb = (g + 1) % 2
            pltpu.async_copy(
                src_hbm.at[pl.ds(base + (g + 1) * grp, grp)],
                sidx.at[nb, pl.ds(0, grp)], sem_i)
            pltpu.async_copy(
                dst_hbm.at[pl.ds(base + (g + 1) * grp, grp)],
                didx.at[nb, pl.ds(0, grp)], sem_i)

        @pl.loop(0, grp, step=2)
        def _pipe(j):
            pltpu.async_copy(tab_hbm.at[sidx.at[bg, j + 1]], rows1, sem_g1)
            pltpu.make_async_copy(
                tab_hbm.at[sidx.at[bg, j]], rows0, sem_g0).wait()
            pltpu.sync_copy(rows0, acc.at[didx.at[bg, j]], add=True)

            @pl.when(j + 2 < grp)
            def _():
                pltpu.async_copy(
                    tab_hbm.at[sidx.at[bg, j + 2]], rows0, sem_g0)

            pltpu.make_async_copy(
                tab_hbm.at[sidx.at[bg, j + 1]], rows1, sem_g1).wait()
            pltpu.sync_copy(rows1, acc.at[didx.at[bg, j + 1]], add=True)

        if g + 1 < _NGRP:
            nb = (g + 1) % 2
            pltpu.make_async_copy(
                src_hbm.at[pl.ds(base + (g + 1) * grp, grp)],
                sidx.at[nb, pl.ds(0, grp)], sem_i).wait()
            pltpu.make_async_copy(
                dst_hbm.at[pl.ds(base + (g + 1) * grp, grp)],
                didx.at[nb, pl.ds(0, grp)], sem_i).wait()
            pltpu.async_copy(tab_hbm.at[sidx.at[nb, 0]], rows0, sem_g0)


def _spmm_body(tab_hbm, src_hbm, dst_hbm, out_hbm,
               sidx, didx, rows0, rows1, acc,
               sem_i, sem_g0, sem_g1):
    c = lax.axis_index("c")
    s = lax.axis_index("s")

    @pl.loop(0, _CHUNK)
    def _fill(i):
        for k in range(_D // 16):
            rows0[i, pl.ds(k * 16, 16)] = jnp.zeros((16,), jnp.float32)

    for k in range(_RPT // _CHUNK):
        pltpu.sync_copy(rows0, acc.at[pl.ds(s * _RPT + k * _CHUNK, _CHUNK)])

    plsc.subcore_barrier()

    @pl.when(c == 0)
    def _core0():
        _edge_pipeline(tab_hbm, src_hbm, dst_hbm, sidx, didx, rows0, rows1,
                       acc, sem_i, sem_g0, sem_g1,
                       base=s * _C0_TPT, grp=_C0_GRP)

    @pl.when(c == 1)
    def _core1():
        _edge_pipeline(tab_hbm, src_hbm, dst_hbm, sidx, didx, rows0, rows1,
                       acc, sem_i, sem_g0, sem_g1,
                       base=16 * _C0_TPT + s * _C1_TPT, grp=_C1_GRP)

    plsc.subcore_barrier()
    pltpu.sync_copy(acc.at[pl.ds(s * _RPT, _RPT)],
                    out_hbm.at[c, pl.ds(s * _RPT, _RPT)])


@functools.cache
def _spmm_kernel():
    return pl.kernel(
        _spmm_body,
        out_type=jax.ShapeDtypeStruct((2, _NPAD, _D), jnp.float32),
        mesh=_mesh(),
        scratch_types=[
            pltpu.VMEM((2, _GRPMAX, _CHUNK), jnp.int32),
            pltpu.VMEM((2, _GRPMAX, _CHUNK), jnp.int32),
            pltpu.VMEM((_CHUNK, _D), jnp.float32),
            pltpu.VMEM((_CHUNK, _D), jnp.float32),
            pltpu.VMEM_SHARED((_NPAD, _D), jnp.float32),
            pltpu.SemaphoreType.DMA,
            pltpu.SemaphoreType.DMA,
            pltpu.SemaphoreType.DMA,
        ],
    )


def _spmm(tab, srcp, dstp):
    return _spmm_kernel()(tab, srcp, dstp)


_BN = 2000


def _tca_body(x_ref, w1_ref, d0_ref, d1_ref, y_ref, dis_ref):
    xw = jnp.dot(x_ref[...], w1_ref[...], preferred_element_type=jnp.float32)
    dis = lax.rsqrt(d0_ref[...] + d1_ref[...] + 1.0)
    y_ref[...] = xw * dis
    dis_ref[...] = dis


def _tca(x, w1, d0, d1):
    return pl.pallas_call(
        _tca_body,
        grid=(_N // _BN,),
        in_specs=[
            pl.BlockSpec((_BN, _D), lambda i: (i, 0)),
            pl.BlockSpec((_D, _D), lambda i: (0, 0)),
            pl.BlockSpec((_BN, 1), lambda i: (i, 0)),
            pl.BlockSpec((_BN, 1), lambda i: (i, 0)),
        ],
        out_specs=[
            pl.BlockSpec((_BN, _D), lambda i: (i, 0)),
            pl.BlockSpec((_BN, 1), lambda i: (i, 0)),
        ],
        out_shape=[
            jax.ShapeDtypeStruct((_N, _D), jnp.float32),
            jax.ShapeDtypeStruct((_N, 1), jnp.float32),
        ],
    )(x, w1, d0, d1)


def _tcb_body(s_ref, y_ref, dis_ref, b1_ref, y2_ref):
    dis = dis_ref[...]
    h = jnp.maximum(dis * (s_ref[0] + s_ref[1] + y_ref[...]) + b1_ref[...],
                    0.0)
    y2_ref[...] = dis * h


def _tcb(s1, y, dis, b1):
    return pl.pallas_call(
        _tcb_body,
        grid=(_N // _BN,),
        in_specs=[
            pl.BlockSpec((2, _BN, _D), lambda i: (0, i, 0)),
            pl.BlockSpec((_BN, _D), lambda i: (i, 0)),
            pl.BlockSpec((_BN, 1), lambda i: (i, 0)),
            pl.BlockSpec((1, _D), lambda i: (0, 0)),
        ],
        out_specs=pl.BlockSpec((_BN, _D), lambda i: (i, 0)),
        out_shape=jax.ShapeDtypeStruct((_N, _D), jnp.float32),
    )(s1, y, dis, b1)


def _tcc_body(s_ref, y2_ref, dis_ref, wmu_ref, bmu_ref, wls_ref, bls_ref,
              mu_ref, ls_ref):
    agg = dis_ref[...] * (s_ref[0] + s_ref[1] + y2_ref[...])
    mu_ref[...] = (jnp.dot(agg, wmu_ref[...],
                           preferred_element_type=jnp.float32) + bmu_ref[...])
    ls_ref[...] = (jnp.dot(agg, wls_ref[...],
                           preferred_element_type=jnp.float32) + bls_ref[...])


def _tcc(s2, y2, dis, wmu, bmu, wls, bls):
    return pl.pallas_call(
        _tcc_body,
        grid=(_N // _BN,),
        in_specs=[
            pl.BlockSpec((2, _BN, _D), lambda i: (0, i, 0)),
            pl.BlockSpec((_BN, _D), lambda i: (i, 0)),
            pl.BlockSpec((_BN, 1), lambda i: (i, 0)),
            pl.BlockSpec((_D, _DO), lambda i: (0, 0)),
            pl.BlockSpec((1, _DO), lambda i: (0, 0)),
            pl.BlockSpec((_D, _DO), lambda i: (0, 0)),
            pl.BlockSpec((1, _DO), lambda i: (0, 0)),
        ],
        out_specs=[
            pl.BlockSpec((_BN, _DO), lambda i: (i, 0)),
            pl.BlockSpec((_BN, _DO), lambda i: (i, 0)),
        ],
        out_shape=[
            jax.ShapeDtypeStruct((_N, _DO), jnp.float32),
            jax.ShapeDtypeStruct((_N, _DO), jnp.float32),
        ],
    )(s2, y2, dis, wmu, bmu, wls, bls)


def kernel(x, edge_index, W1, b1, Wmu, bmu, Wls, bls):
    src = edge_index[0]
    dst = edge_index[1]
    pad = _EPAD - _E
    srcp = jnp.concatenate(
        [src, jnp.zeros((pad,), jnp.int32)]).reshape(_EROWS, _CHUNK)
    dstp = jnp.concatenate(
        [dst, _N + (jnp.arange(pad, dtype=jnp.int32) % (_NPAD - _N))]
    ).reshape(_EROWS, _CHUNK)

    dcnt = _deg(dstp)
    d0 = dcnt[0, :_N].reshape(_N, 1)
    d1 = dcnt[1, :_N].reshape(_N, 1)

    y, dis = _tca(x, W1, d0, d1)
    s1 = _spmm(y, srcp, dstp)[:, :_N, :]
    y2 = _tcb(s1, y, dis, b1.reshape(1, _D))
    s2 = _spmm(y2, srcp, dstp)[:, :_N, :]
    mu, ls = _tcc(s2, y2, dis, Wmu, bmu.reshape(1, _DO), Wls,
                  bls.reshape(1, _DO))
    return (mu, ls)
